# baseline (speedup 1.0000x reference)
import jax
import jax.numpy as jnp
from jax import lax
from jax.experimental import pallas as pl
from jax.experimental.pallas import tpu as pltpu
from jax.experimental.pallas import tpu_sc as plsc

_N = 10000
_E = 320000
_D = 128
_H = 128
_G = 64

_NC = 2
_NS = 16
_NW = _NC * _NS
_CH = 80
_EPW = _E // _NW
_NCHUNK = _EPW // _CH
_NP = 10240
_RPT = _NP // _NS
_ZB = 128

_mesh = plsc.VectorSubcoreMesh(
    core_axis_name="c", subcore_axis_name="s", num_cores=_NC, num_subcores=_NS
)


def _sc_deg_body(dstr_hbm, out_hbm, didx_all, ones_v, zbuf, deg_sh, ssem):
    c = lax.axis_index("c")
    s = lax.axis_index("s")
    w = c * _NS + s

    pltpu.sync_copy(dstr_hbm.at[w], didx_all)

    def _fill_ones(i, carry):
        ones_v[i, :] = jnp.ones((16,), jnp.float32)
        return carry

    lax.fori_loop(0, _CH, _fill_ones, 0)

    def _fill_zero(i, carry):
        zbuf[i, :] = jnp.zeros((16,), jnp.float32)
        return carry

    lax.fori_loop(0, _ZB, _fill_zero, 0)

    def _zero_copy(k, carry):
        pltpu.sync_copy(zbuf, deg_sh.at[pl.ds(s * _RPT + k * _ZB, _ZB)])
        return carry

    lax.fori_loop(0, _RPT // _ZB, _zero_copy, 0)
    plsc.subcore_barrier()

    pltpu.async_copy(ones_v, deg_sh.at[didx_all.at[0]], ssem, add=True)

    def _chunk(i, carry):
        pltpu.make_async_copy(ones_v, deg_sh.at[didx_all.at[i - 1]],
                              ssem).wait()
        pltpu.async_copy(ones_v, deg_sh.at[didx_all.at[i]], ssem, add=True)
        return carry

    lax.fori_loop(1, _NCHUNK, _chunk, 0)
    pltpu.make_async_copy(ones_v, deg_sh.at[didx_all.at[_NCHUNK - 1]],
                          ssem).wait()
    plsc.subcore_barrier()
    pltpu.sync_copy(deg_sh.at[pl.ds(s * _RPT, _RPT)],
                    out_hbm.at[c, pl.ds(s * _RPT, _RPT)])


_sc_deg = pl.kernel(
    _sc_deg_body,
    out_type=jax.ShapeDtypeStruct((_NC, _NP, 16), jnp.float32),
    mesh=_mesh,
    scratch_types=[
        pltpu.VMEM((_NCHUNK, _CH), jnp.int32),
        pltpu.VMEM((_CH, 16), jnp.float32),
        pltpu.VMEM((_ZB, 16), jnp.float32),
        pltpu.VMEM_SHARED((_NP, 16), jnp.float32),
        pltpu.SemaphoreType.DMA,
    ],
)


def _sc_agg_body(g_hbm, src_hbm, dst_hbm, out_hbm,
                 sidx_all, didx, rows, agg_sh, gsem, dsem, ssem):
    c = lax.axis_index("c")
    s = lax.axis_index("s")
    w = c * _NS + s
    _last = _NCHUNK - 1

    pltpu.sync_copy(src_hbm.at[pl.ds(w * _EPW, _EPW)], sidx_all)

    def _fill_zero(i, carry):
        for j in range(_H // 16):
            rows[0, i, pl.ds(j * 16, 16)] = jnp.zeros((16,), jnp.float32)
        return carry

    lax.fori_loop(0, _CH, _fill_zero, 0)

    def _zero_copy(k, carry):
        pltpu.sync_copy(rows.at[0], agg_sh.at[pl.ds(s * _RPT + k * _CH, _CH)])
        return carry

    lax.fori_loop(0, _RPT // _CH, _zero_copy, 0)
    plsc.subcore_barrier()

    def _gather(i, b):
        pltpu.async_copy(g_hbm.at[sidx_all.at[pl.ds(i * _CH, _CH)]],
                         rows.at[b], gsem.at[b])

    def _gwait(i, b):
        pltpu.make_async_copy(g_hbm.at[sidx_all.at[pl.ds(i * _CH, _CH)]],
                              rows.at[b], gsem.at[b]).wait()

    def _dstage(i, b):
        pltpu.async_copy(dst_hbm.at[pl.ds(w * _EPW + i * _CH, _CH)],
                         didx.at[b], dsem.at[b])

    def _dwait(i, b):
        pltpu.make_async_copy(dst_hbm.at[pl.ds(w * _EPW + i * _CH, _CH)],
                              didx.at[b], dsem.at[b]).wait()

    def _sstart(b):
        pltpu.async_copy(rows.at[b], agg_sh.at[didx.at[b]], ssem.at[b],
                         add=True)

    def _swait(b):
        pltpu.make_async_copy(rows.at[b], agg_sh.at[didx.at[b]],
                              ssem.at[b]).wait()

    _dstage(0, 0)
    _dstage(1, 1)
    _gather(0, 0)
    _gather(1, 1)
    _gwait(0, 0)
    _dwait(0, 0)
    _sstart(0)
    _dstage(2, 2)
    _gather(2, 2)

    def _step(j, b, bp):
        jn = jnp.minimum(j + 2, _last)
        _gwait(j, b)
        _dwait(j, b)
        _swait(bp)
        _sstart(b)
        _dstage(jn, bp)
        _gather(jn, bp)

    def _iter(m, carry):
        j = 3 * m + 1
        _step(j, 1, 0)
        _step(j + 1, 2, 1)
        _step(j + 2, 0, 2)
        return carry

    lax.fori_loop(0, (_NCHUNK - 2) // 3, _iter, 0)
    _gwait(_last, 1)
    _dwait(_last, 1)
    _swait(0)
    _sstart(1)
    _gwait(_last, 2)
    _dwait(_last, 2)
    _swait(1)

    plsc.subcore_barrier()
    pltpu.sync_copy(agg_sh.at[pl.ds(s * _RPT, _RPT)],
                    out_hbm.at[c, pl.ds(s * _RPT, _RPT)])


_sc_agg = pl.kernel(
    _sc_agg_body,
    out_type=jax.ShapeDtypeStruct((_NC, _NP, _H), jnp.float32),
    mesh=_mesh,
    scratch_types=[
        pltpu.VMEM((_EPW,), jnp.int32),
        pltpu.VMEM((3, _CH), jnp.int32),
        pltpu.VMEM((3, _CH, _H), jnp.float32),
        pltpu.VMEM_SHARED((_NP, _H), jnp.float32),
        pltpu.SemaphoreType.DMA((3,)),
        pltpu.SemaphoreType.DMA((3,)),
        pltpu.SemaphoreType.DMA((3,)),
    ],
)


_BM = 1000


def _tc1_body(x_ref, w_ref, deg_ref, dinv_ref, g_ref):
    d = deg_ref[0, :, :8] + deg_ref[1, :, :8] + 1.0
    dinv8 = lax.rsqrt(d)
    dinv_ref[...] = dinv8
    h = jnp.dot(x_ref[...], w_ref[...], preferred_element_type=jnp.float32)
    g_ref[...] = dinv8[:, :1] * h


def _tc1(x, W1, deg):
    return pl.pallas_call(
        _tc1_body,
        grid=(_N // _BM,),
        in_specs=[
            pl.BlockSpec((_BM, _D), lambda i: (i, 0)),
            pl.BlockSpec((_D, _H), lambda i: (0, 0)),
            pl.BlockSpec((_NC, _BM, 16), lambda i: (0, i, 0)),
        ],
        out_specs=[
            pl.BlockSpec((_BM, 8), lambda i: (i, 0)),
            pl.BlockSpec((_BM, _H), lambda i: (i, 0)),
        ],
        out_shape=[
            jax.ShapeDtypeStruct((_N, 8), jnp.float32),
            jax.ShapeDtypeStruct((_N, _H), jnp.float32),
        ],
    )(x, W1, deg)


def _tc_mid_body(s_ref, g_ref, dinv_ref, b_ref, w_ref, gn_ref):
    dv = dinv_ref[:, :1]
    h = dv * (s_ref[0] + s_ref[1] + g_ref[...]) + b_ref[...]
    h = jnp.maximum(h, 0.0)
    gn_ref[...] = dv * jnp.dot(h, w_ref[...], preferred_element_type=jnp.float32)


def _tc_mid(s, g, dinv, b, Wn):
    return pl.pallas_call(
        _tc_mid_body,
        grid=(_N // _BM,),
        in_specs=[
            pl.BlockSpec((_NC, _BM, _H), lambda i: (0, i, 0)),
            pl.BlockSpec((_BM, _H), lambda i: (i, 0)),
            pl.BlockSpec((_BM, 8), lambda i: (i, 0)),
            pl.BlockSpec((1, _H), lambda i: (0, 0)),
            pl.BlockSpec((_H, _H), lambda i: (0, 0)),
        ],
        out_specs=pl.BlockSpec((_BM, _H), lambda i: (i, 0)),
        out_shape=jax.ShapeDtypeStruct((_N, _H), jnp.float32),
    )(s, g, dinv, b, Wn)


def _tc_final_body(s_ref, g_ref, dinv_ref, b_ref, batch_ref, out_ref,
                   pooled_acc, cnt_acc):
    i = pl.program_id(0)

    @pl.when(i == 0)
    def _():
        pooled_acc[...] = jnp.zeros_like(pooled_acc)
        cnt_acc[...] = jnp.zeros_like(cnt_acc)

    dv = dinv_ref[:, :1]
    h = dv * (s_ref[0] + s_ref[1] + g_ref[...]) + b_ref[...]
    bt = batch_ref[0]
    gid = lax.broadcasted_iota(jnp.int32, (_G, _BM), 0)
    oh = (bt == gid).astype(jnp.float32)
    pooled_acc[...] += jnp.dot(oh, h, preferred_element_type=jnp.float32)
    cnt_acc[...] += jnp.sum(oh, axis=1, keepdims=True)

    @pl.when(i == pl.num_programs(0) - 1)
    def _():
        out_ref[...] = pooled_acc[...] / jnp.maximum(cnt_acc[...], 1.0)


def _tc_final(s, g, dinv, b, batch3):
    return pl.pallas_call(
        _tc_final_body,
        grid=(_N // _BM,),
        in_specs=[
            pl.BlockSpec((_NC, _BM, _H), lambda i: (0, i, 0)),
            pl.BlockSpec((_BM, _H), lambda i: (i, 0)),
            pl.BlockSpec((_BM, 8), lambda i: (i, 0)),
            pl.BlockSpec((1, _H), lambda i: (0, 0)),
            pl.BlockSpec((1, 1, _BM), lambda i: (i, 0, 0)),
        ],
        out_specs=pl.BlockSpec((_G, _H), lambda i: (0, 0)),
        out_shape=jax.ShapeDtypeStruct((_G, _H), jnp.float32),
        scratch_shapes=[
            pltpu.VMEM((_G, _H), jnp.float32),
            pltpu.VMEM((_G, _H), jnp.float32),
        ],
        compiler_params=pltpu.CompilerParams(
            dimension_semantics=("arbitrary",)),
    )(s, g, dinv, b, batch3)


def kernel(x, edge_index, batch, W1, b1, W2, b2, W3, b3):
    src = edge_index[0]
    dst = edge_index[1]
    dstr = dst.reshape(_NW, _NCHUNK, _CH)
    batch3 = batch.reshape(_N // _BM, 1, _BM)
    b1r = b1.reshape(1, _H)
    b2r = b2.reshape(1, _H)
    b3r = b3.reshape(1, _H)

    deg = _sc_deg(dstr)
    dinv, g1 = _tc1(x, W1, deg)
    s1 = _sc_agg(g1, src, dst)
    g2 = _tc_mid(s1, g1, dinv, b1r, W2)
    s2 = _sc_agg(g2, src, dst)
    g3 = _tc_mid(s2, g2, dinv, b2r, W3)
    s3 = _sc_agg(g3, src, dst)
    return _tc_final(s3, g3, dinv, b3r, batch3)


# speedup vs baseline: 1.0239x; 1.0239x over previous
import jax
import jax.numpy as jnp
from jax import lax
from jax.experimental import pallas as pl
from jax.experimental.pallas import tpu as pltpu
from jax.experimental.pallas import tpu_sc as plsc

_N = 10000
_E = 320000
_D = 128
_H = 128
_G = 64

_NC = 2
_NS = 16
_NW = _NC * _NS
_CH = 80
_EPW = _E // _NW
_NCHUNK = _EPW // _CH
_NP = 10240
_RPT = _NP // _NS
_ZB = 128

_mesh = plsc.VectorSubcoreMesh(
    core_axis_name="c", subcore_axis_name="s", num_cores=_NC, num_subcores=_NS
)


def _sc_deg_body(dstr_hbm, out_hbm, didx_all, ones_v, zbuf, deg_sh):
    c = lax.axis_index("c")
    s = lax.axis_index("s")
    w = c * _NS + s

    pltpu.sync_copy(dstr_hbm.at[w], didx_all)

    def _fill_ones(i, carry):
        ones_v[i, :] = jnp.ones((16,), jnp.float32)
        return carry

    lax.fori_loop(0, _CH, _fill_ones, 0)

    def _fill_zero(i, carry):
        zbuf[i, :] = jnp.zeros((16,), jnp.float32)
        return carry

    lax.fori_loop(0, _ZB, _fill_zero, 0)

    def _zero_copy(k, carry):
        pltpu.sync_copy(zbuf, deg_sh.at[pl.ds(s * _RPT + k * _ZB, _ZB)])
        return carry

    lax.fori_loop(0, _RPT // _ZB, _zero_copy, 0)
    plsc.subcore_barrier()

    def _chunk(i, carry):
        pltpu.sync_copy(ones_v, deg_sh.at[didx_all.at[i]], add=True)
        return carry

    lax.fori_loop(0, _NCHUNK, _chunk, 0)
    plsc.subcore_barrier()
    pltpu.sync_copy(deg_sh.at[pl.ds(s * _RPT, _RPT)],
                    out_hbm.at[c, pl.ds(s * _RPT, _RPT)])


_sc_deg = pl.kernel(
    _sc_deg_body,
    out_type=jax.ShapeDtypeStruct((_NC, _NP, 16), jnp.float32),
    mesh=_mesh,
    scratch_types=[
        pltpu.VMEM((_NCHUNK, _CH), jnp.int32),
        pltpu.VMEM((_CH, 16), jnp.float32),
        pltpu.VMEM((_ZB, 16), jnp.float32),
        pltpu.VMEM_SHARED((_NP, 16), jnp.float32),
    ],
)


def _sc_agg_body(g_hbm, src_hbm, dst_hbm, out_hbm,
                 sidx_all, didx, rows, agg_sh, gsem, dsem, ssem):
    c = lax.axis_index("c")
    s = lax.axis_index("s")
    w = c * _NS + s
    _last = _NCHUNK - 1

    pltpu.sync_copy(src_hbm.at[pl.ds(w * _EPW, _EPW)], sidx_all)

    def _fill_zero(i, carry):
        for j in range(_H // 16):
            rows[0, i, pl.ds(j * 16, 16)] = jnp.zeros((16,), jnp.float32)
        return carry

    lax.fori_loop(0, _CH, _fill_zero, 0)

    def _zero_copy(k, carry):
        pltpu.sync_copy(rows.at[0], agg_sh.at[pl.ds(s * _RPT + k * _CH, _CH)])
        return carry

    lax.fori_loop(0, _RPT // _CH, _zero_copy, 0)
    plsc.subcore_barrier()

    def _gather(i, b):
        pltpu.async_copy(g_hbm.at[sidx_all.at[pl.ds(i * _CH, _CH)]],
                         rows.at[b], gsem.at[b])

    def _gwait(i, b):
        pltpu.make_async_copy(g_hbm.at[sidx_all.at[pl.ds(i * _CH, _CH)]],
                              rows.at[b], gsem.at[b]).wait()

    def _dstage(i, b):
        pltpu.async_copy(dst_hbm.at[pl.ds(w * _EPW + i * _CH, _CH)],
                         didx.at[b], dsem.at[b])

    def _dwait(i, b):
        pltpu.make_async_copy(dst_hbm.at[pl.ds(w * _EPW + i * _CH, _CH)],
                              didx.at[b], dsem.at[b]).wait()

    def _sstart(b):
        pltpu.async_copy(rows.at[b], agg_sh.at[didx.at[b]], ssem.at[b],
                         add=True)

    def _swait(b):
        pltpu.make_async_copy(rows.at[b], agg_sh.at[didx.at[b]],
                              ssem.at[b]).wait()

    _dstage(0, 0)
    _dstage(1, 1)
    _gather(0, 0)
    _gather(1, 1)
    _gwait(0, 0)
    _dwait(0, 0)
    _sstart(0)
    _dstage(2, 2)
    _gather(2, 2)

    def _step(j, b, bp):
        jn = jnp.minimum(j + 2, _last)
        _gwait(j, b)
        _dwait(j, b)
        _swait(bp)
        _sstart(b)
        _dstage(jn, bp)
        _gather(jn, bp)

    def _iter(m, carry):
        j = 3 * m + 1
        _step(j, 1, 0)
        _step(j + 1, 2, 1)
        _step(j + 2, 0, 2)
        return carry

    lax.fori_loop(0, (_NCHUNK - 2) // 3, _iter, 0)
    _gwait(_last, 1)
    _dwait(_last, 1)
    _swait(0)
    _sstart(1)
    _gwait(_last, 2)
    _dwait(_last, 2)
    _swait(1)

    plsc.subcore_barrier()
    pltpu.sync_copy(agg_sh.at[pl.ds(s * _RPT, _RPT)],
                    out_hbm.at[c, pl.ds(s * _RPT, _RPT)])


_sc_agg = pl.kernel(
    _sc_agg_body,
    out_type=jax.ShapeDtypeStruct((_NC, _NP, _H), jnp.float32),
    mesh=_mesh,
    scratch_types=[
        pltpu.VMEM((_EPW,), jnp.int32),
        pltpu.VMEM((3, _CH), jnp.int32),
        pltpu.VMEM((3, _CH, _H), jnp.float32),
        pltpu.VMEM_SHARED((_NP, _H), jnp.float32),
        pltpu.SemaphoreType.DMA((3,)),
        pltpu.SemaphoreType.DMA((3,)),
        pltpu.SemaphoreType.DMA((3,)),
    ],
)


_BM = 2000


def _tc1_body(x_ref, w_ref, deg_ref, dinv_ref, g_ref):
    d = deg_ref[0, :, :8] + deg_ref[1, :, :8] + 1.0
    dinv8 = lax.rsqrt(d)
    dinv_ref[...] = dinv8
    h = jnp.dot(x_ref[...], w_ref[...], preferred_element_type=jnp.float32)
    g_ref[...] = dinv8[:, :1] * h


def _tc1(x, W1, deg):
    return pl.pallas_call(
        _tc1_body,
        grid=(_N // _BM,),
        in_specs=[
            pl.BlockSpec((_BM, _D), lambda i: (i, 0)),
            pl.BlockSpec((_D, _H), lambda i: (0, 0)),
            pl.BlockSpec((_NC, _BM, 16), lambda i: (0, i, 0)),
        ],
        out_specs=[
            pl.BlockSpec((_BM, 8), lambda i: (i, 0)),
            pl.BlockSpec((_BM, _H), lambda i: (i, 0)),
        ],
        out_shape=[
            jax.ShapeDtypeStruct((_N, 8), jnp.float32),
            jax.ShapeDtypeStruct((_N, _H), jnp.float32),
        ],
    )(x, W1, deg)


def _tc_mid_body(s_ref, g_ref, dinv_ref, b_ref, w_ref, gn_ref):
    dv = dinv_ref[:, :1]
    h = dv * (s_ref[0] + s_ref[1] + g_ref[...]) + b_ref[...]
    h = jnp.maximum(h, 0.0)
    gn_ref[...] = dv * jnp.dot(h, w_ref[...], preferred_element_type=jnp.float32)


def _tc_mid(s, g, dinv, b, Wn):
    return pl.pallas_call(
        _tc_mid_body,
        grid=(_N // _BM,),
        in_specs=[
            pl.BlockSpec((_NC, _BM, _H), lambda i: (0, i, 0)),
            pl.BlockSpec((_BM, _H), lambda i: (i, 0)),
            pl.BlockSpec((_BM, 8), lambda i: (i, 0)),
            pl.BlockSpec((1, _H), lambda i: (0, 0)),
            pl.BlockSpec((_H, _H), lambda i: (0, 0)),
        ],
        out_specs=pl.BlockSpec((_BM, _H), lambda i: (i, 0)),
        out_shape=jax.ShapeDtypeStruct((_N, _H), jnp.float32),
    )(s, g, dinv, b, Wn)


def _tc_final_body(s_ref, g_ref, dinv_ref, b_ref, batch_ref, out_ref,
                   pooled_acc, cnt_acc):
    i = pl.program_id(0)

    @pl.when(i == 0)
    def _():
        pooled_acc[...] = jnp.zeros_like(pooled_acc)
        cnt_acc[...] = jnp.zeros_like(cnt_acc)

    dv = dinv_ref[:, :1]
    h = dv * (s_ref[0] + s_ref[1] + g_ref[...]) + b_ref[...]
    bt = batch_ref[0]
    gid = lax.broadcasted_iota(jnp.int32, (_G, _BM), 0)
    oh = (bt == gid).astype(jnp.float32)
    pooled_acc[...] += jnp.dot(oh, h, preferred_element_type=jnp.float32)
    cnt_acc[...] += jnp.sum(oh, axis=1, keepdims=True)

    @pl.when(i == pl.num_programs(0) - 1)
    def _():
        out_ref[...] = pooled_acc[...] / jnp.maximum(cnt_acc[...], 1.0)


def _tc_final(s, g, dinv, b, batch3):
    return pl.pallas_call(
        _tc_final_body,
        grid=(_N // _BM,),
        in_specs=[
            pl.BlockSpec((_NC, _BM, _H), lambda i: (0, i, 0)),
            pl.BlockSpec((_BM, _H), lambda i: (i, 0)),
            pl.BlockSpec((_BM, 8), lambda i: (i, 0)),
            pl.BlockSpec((1, _H), lambda i: (0, 0)),
            pl.BlockSpec((1, 1, _BM), lambda i: (i, 0, 0)),
        ],
        out_specs=pl.BlockSpec((_G, _H), lambda i: (0, 0)),
        out_shape=jax.ShapeDtypeStruct((_G, _H), jnp.float32),
        scratch_shapes=[
            pltpu.VMEM((_G, _H), jnp.float32),
            pltpu.VMEM((_G, _H), jnp.float32),
        ],
        compiler_params=pltpu.CompilerParams(
            dimension_semantics=("arbitrary",)),
    )(s, g, dinv, b, batch3)


def kernel(x, edge_index, batch, W1, b1, W2, b2, W3, b3):
    src = edge_index[0]
    dst = edge_index[1]
    dstr = dst.reshape(_NW, _NCHUNK, _CH)
    batch3 = batch.reshape(_N // _BM, 1, _BM)
    b1r = b1.reshape(1, _H)
    b2r = b2.reshape(1, _H)
    b3r = b3.reshape(1, _H)

    deg = _sc_deg(dstr)
    dinv, g1 = _tc1(x, W1, deg)
    s1 = _sc_agg(g1, src, dst)
    g2 = _tc_mid(s1, g1, dinv, b1r, W2)
    s2 = _sc_agg(g2, src, dst)
    g3 = _tc_mid(s2, g2, dinv, b2r, W3)
    s3 = _sc_agg(g3, src, dst)
    return _tc_final(s3, g3, dinv, b3r, batch3)


# speedup vs baseline: 1.0311x; 1.0070x over previous
import jax
import jax.numpy as jnp
from jax import lax
from jax.experimental import pallas as pl
from jax.experimental.pallas import tpu as pltpu
from jax.experimental.pallas import tpu_sc as plsc

_N = 10000
_E = 320000
_D = 128
_H = 128
_G = 64

_NC = 2
_NS = 16
_NW = _NC * _NS
_CH = 80
_EPW = _E // _NW
_NCHUNK = _EPW // _CH
_NP = 10240
_RPT = _NP // _NS
_ZB = 128

_mesh = plsc.VectorSubcoreMesh(
    core_axis_name="c", subcore_axis_name="s", num_cores=_NC, num_subcores=_NS
)


def _sc_deg_body(dstr_hbm, out_hbm, didx_all, ones_v, zbuf, deg_sh):
    c = lax.axis_index("c")
    s = lax.axis_index("s")
    w = c * _NS + s

    pltpu.sync_copy(dstr_hbm.at[w], didx_all)

    def _fill_ones(i, carry):
        ones_v[i, :] = jnp.ones((16,), jnp.float32)
        return carry

    lax.fori_loop(0, _CH, _fill_ones, 0)

    def _fill_zero(i, carry):
        zbuf[i, :] = jnp.zeros((16,), jnp.float32)
        return carry

    lax.fori_loop(0, _ZB, _fill_zero, 0)

    def _zero_copy(k, carry):
        pltpu.sync_copy(zbuf, deg_sh.at[pl.ds(s * _RPT + k * _ZB, _ZB)])
        return carry

    lax.fori_loop(0, _RPT // _ZB, _zero_copy, 0)
    plsc.subcore_barrier()

    def _chunk(i, carry):
        pltpu.sync_copy(ones_v, deg_sh.at[didx_all.at[i]], add=True)
        return carry

    lax.fori_loop(0, _NCHUNK, _chunk, 0)
    plsc.subcore_barrier()
    pltpu.sync_copy(deg_sh.at[pl.ds(s * _RPT, _RPT)],
                    out_hbm.at[c, pl.ds(s * _RPT, _RPT)])


_sc_deg = pl.kernel(
    _sc_deg_body,
    out_type=jax.ShapeDtypeStruct((_NC, _NP, 16), jnp.float32),
    mesh=_mesh,
    scratch_types=[
        pltpu.VMEM((_NCHUNK, _CH), jnp.int32),
        pltpu.VMEM((_CH, 16), jnp.float32),
        pltpu.VMEM((_ZB, 16), jnp.float32),
        pltpu.VMEM_SHARED((_NP, 16), jnp.float32),
    ],
)


def _sc_agg_body(g_hbm, src_hbm, dst_hbm, out_hbm,
                 sidx_all, didx, rows, agg_sh, gsem, dsem, ssem):
    c = lax.axis_index("c")
    s = lax.axis_index("s")
    w = c * _NS + s
    _last = _NCHUNK - 1

    pltpu.sync_copy(src_hbm.at[pl.ds(w * _EPW, _EPW)], sidx_all)

    def _fill_zero(i, carry):
        for j in range(_H // 16):
            rows[0, i, pl.ds(j * 16, 16)] = jnp.zeros((16,), jnp.float32)
        return carry

    lax.fori_loop(0, _CH, _fill_zero, 0)

    def _zero_copy(k, carry):
        pltpu.sync_copy(rows.at[0], agg_sh.at[pl.ds(s * _RPT + k * _CH, _CH)])
        return carry

    lax.fori_loop(0, _RPT // _CH, _zero_copy, 0)
    plsc.subcore_barrier()

    def _gather(i, b):
        pltpu.async_copy(g_hbm.at[sidx_all.at[pl.ds(i * _CH, _CH)]],
                         rows.at[b], gsem.at[b])

    def _gwait(i, b):
        pltpu.make_async_copy(g_hbm.at[sidx_all.at[pl.ds(i * _CH, _CH)]],
                              rows.at[b], gsem.at[b]).wait()

    def _dstage(i, b):
        pltpu.async_copy(dst_hbm.at[pl.ds(w * _EPW + i * _CH, _CH)],
                         didx.at[b], dsem.at[b])

    def _dwait(i, b):
        pltpu.make_async_copy(dst_hbm.at[pl.ds(w * _EPW + i * _CH, _CH)],
                              didx.at[b], dsem.at[b]).wait()

    def _sstart(b):
        pltpu.async_copy(rows.at[b], agg_sh.at[didx.at[b]], ssem.at[b],
                         add=True)

    def _swait(b):
        pltpu.make_async_copy(rows.at[b], agg_sh.at[didx.at[b]],
                              ssem.at[b]).wait()

    _dstage(0, 0)
    _dstage(1, 1)
    _gather(0, 0)
    _gather(1, 1)
    _gwait(0, 0)
    _dwait(0, 0)
    _sstart(0)
    _dstage(2, 2)
    _gather(2, 2)

    def _step(j, b, bp):
        jn = jnp.minimum(j + 2, _last)
        _gwait(j, b)
        _dwait(j, b)
        _swait(bp)
        _sstart(b)
        _dstage(jn, bp)
        _gather(jn, bp)

    def _iter(m, carry):
        j = 3 * m + 1
        _step(j, 1, 0)
        _step(j + 1, 2, 1)
        _step(j + 2, 0, 2)
        return carry

    lax.fori_loop(0, (_NCHUNK - 2) // 3, _iter, 0)
    _gwait(_last, 1)
    _dwait(_last, 1)
    _swait(0)
    _sstart(1)
    _gwait(_last, 2)
    _dwait(_last, 2)
    _swait(1)

    plsc.subcore_barrier()
    pltpu.sync_copy(agg_sh.at[pl.ds(s * _RPT, _RPT)],
                    out_hbm.at[c, pl.ds(s * _RPT, _RPT)])


_sc_agg = pl.kernel(
    _sc_agg_body,
    out_type=jax.ShapeDtypeStruct((_NC, _NP, _H), jnp.float32),
    mesh=_mesh,
    scratch_types=[
        pltpu.VMEM((_EPW,), jnp.int32),
        pltpu.VMEM((3, _CH), jnp.int32),
        pltpu.VMEM((3, _CH, _H), jnp.float32),
        pltpu.VMEM_SHARED((_NP, _H), jnp.float32),
        pltpu.SemaphoreType.DMA((3,)),
        pltpu.SemaphoreType.DMA((3,)),
        pltpu.SemaphoreType.DMA((3,)),
    ],
)


_BM = 5000


def _tc1_body(x_ref, w_ref, deg_ref, dinv_ref, g_ref):
    d = deg_ref[0, :, :8] + deg_ref[1, :, :8] + 1.0
    dinv8 = lax.rsqrt(d)
    dinv_ref[...] = dinv8
    h = jnp.dot(x_ref[...], w_ref[...], preferred_element_type=jnp.float32)
    g_ref[...] = dinv8[:, :1] * h


def _tc1(x, W1, deg):
    return pl.pallas_call(
        _tc1_body,
        grid=(_N // _BM,),
        in_specs=[
            pl.BlockSpec((_BM, _D), lambda i: (i, 0)),
            pl.BlockSpec((_D, _H), lambda i: (0, 0)),
            pl.BlockSpec((_NC, _BM, 16), lambda i: (0, i, 0)),
        ],
        out_specs=[
            pl.BlockSpec((_BM, 8), lambda i: (i, 0)),
            pl.BlockSpec((_BM, _H), lambda i: (i, 0)),
        ],
        out_shape=[
            jax.ShapeDtypeStruct((_N, 8), jnp.float32),
            jax.ShapeDtypeStruct((_N, _H), jnp.float32),
        ],
    )(x, W1, deg)


def _tc_mid_body(s_ref, g_ref, dinv_ref, b_ref, w_ref, gn_ref):
    dv = dinv_ref[:, :1]
    h = dv * (s_ref[0] + s_ref[1] + g_ref[...]) + b_ref[...]
    h = jnp.maximum(h, 0.0)
    gn_ref[...] = dv * jnp.dot(h, w_ref[...], preferred_element_type=jnp.float32)


def _tc_mid(s, g, dinv, b, Wn):
    return pl.pallas_call(
        _tc_mid_body,
        grid=(_N // _BM,),
        in_specs=[
            pl.BlockSpec((_NC, _BM, _H), lambda i: (0, i, 0)),
            pl.BlockSpec((_BM, _H), lambda i: (i, 0)),
            pl.BlockSpec((_BM, 8), lambda i: (i, 0)),
            pl.BlockSpec((1, _H), lambda i: (0, 0)),
            pl.BlockSpec((_H, _H), lambda i: (0, 0)),
        ],
        out_specs=pl.BlockSpec((_BM, _H), lambda i: (i, 0)),
        out_shape=jax.ShapeDtypeStruct((_N, _H), jnp.float32),
    )(s, g, dinv, b, Wn)


def _tc_final_body(s_ref, g_ref, dinv_ref, b_ref, batch_ref, out_ref,
                   pooled_acc, cnt_acc):
    i = pl.program_id(0)

    @pl.when(i == 0)
    def _():
        pooled_acc[...] = jnp.zeros_like(pooled_acc)
        cnt_acc[...] = jnp.zeros_like(cnt_acc)

    dv = dinv_ref[:, :1]
    h = dv * (s_ref[0] + s_ref[1] + g_ref[...]) + b_ref[...]
    bt = batch_ref[0]
    gid = lax.broadcasted_iota(jnp.int32, (_G, _BM), 0)
    oh = (bt == gid).astype(jnp.float32)
    pooled_acc[...] += jnp.dot(oh, h, preferred_element_type=jnp.float32)
    cnt_acc[...] += jnp.sum(oh, axis=1, keepdims=True)

    @pl.when(i == pl.num_programs(0) - 1)
    def _():
        out_ref[...] = pooled_acc[...] / jnp.maximum(cnt_acc[...], 1.0)


def _tc_final(s, g, dinv, b, batch3):
    return pl.pallas_call(
        _tc_final_body,
        grid=(_N // _BM,),
        in_specs=[
            pl.BlockSpec((_NC, _BM, _H), lambda i: (0, i, 0)),
            pl.BlockSpec((_BM, _H), lambda i: (i, 0)),
            pl.BlockSpec((_BM, 8), lambda i: (i, 0)),
            pl.BlockSpec((1, _H), lambda i: (0, 0)),
            pl.BlockSpec((1, 1, _BM), lambda i: (i, 0, 0)),
        ],
        out_specs=pl.BlockSpec((_G, _H), lambda i: (0, 0)),
        out_shape=jax.ShapeDtypeStruct((_G, _H), jnp.float32),
        scratch_shapes=[
            pltpu.VMEM((_G, _H), jnp.float32),
            pltpu.VMEM((_G, _H), jnp.float32),
        ],
        compiler_params=pltpu.CompilerParams(
            dimension_semantics=("arbitrary",)),
    )(s, g, dinv, b, batch3)


def kernel(x, edge_index, batch, W1, b1, W2, b2, W3, b3):
    src = edge_index[0]
    dst = edge_index[1]
    dstr = dst.reshape(_NW, _NCHUNK, _CH)
    batch3 = batch.reshape(_N // _BM, 1, _BM)
    b1r = b1.reshape(1, _H)
    b2r = b2.reshape(1, _H)
    b3r = b3.reshape(1, _H)

    deg = _sc_deg(dstr)
    dinv, g1 = _tc1(x, W1, deg)
    s1 = _sc_agg(g1, src, dst)
    g2 = _tc_mid(s1, g1, dinv, b1r, W2)
    s2 = _sc_agg(g2, src, dst)
    g3 = _tc_mid(s2, g2, dinv, b2r, W3)
    s3 = _sc_agg(g3, src, dst)
    return _tc_final(s3, g3, dinv, b3r, batch3)
